# baseline (speedup 1.0000x reference)
import functools

import jax
import jax.numpy as jnp
from jax import lax
from jax.experimental import pallas as pl
from jax.experimental.pallas import tpu as pltpu
from jax.experimental.pallas import tpu_sc as plsc

NC, NS, L = 2, 16, 16
NW = NC * NS
NROWS, NCOLS = 16384, 1000
ROWS_PER_W = NROWS // NW
RB = 16
NBLK = ROWS_PER_W // RB
CHUNKS = NCOLS // L
CU = 4

_mesh = plsc.VectorSubcoreMesh(core_axis_name="c", subcore_axis_name="s")


@functools.partial(
    pl.kernel,
    out_type=[
        jax.ShapeDtypeStruct((1, NW * L), jnp.float32),
        jax.ShapeDtypeStruct((1, NW * L), jnp.float32),
    ],
    mesh=_mesh,
    scratch_types=[
        pltpu.VMEM((2, RB, NCOLS), jnp.float32),
        pltpu.VMEM((2, RB, NCOLS), jnp.float32),
        pltpu.VMEM((L,), jnp.float32),
        pltpu.VMEM((L,), jnp.float32),
        pltpu.SemaphoreType.DMA,
        pltpu.SemaphoreType.DMA,
        pltpu.SemaphoreType.DMA,
        pltpu.SemaphoreType.DMA,
    ],
)
def _sc_partials(p_hbm, t_hbm, sums_hbm, cnts_hbm,
                 pbuf, tbuf, svec, cvec, sp0, sp1, st0, st1):
    wid = lax.axis_index("s") * NC + lax.axis_index("c")
    row0 = wid * ROWS_PER_W
    sems_p = (sp0, sp1)
    sems_t = (st0, st1)
    lane = lax.iota(jnp.int32, L)
    edge_keep = lane >= (L - (NCOLS - CHUNKS * L))

    def start(i, b):
        r = row0 + i * RB
        pltpu.async_copy(p_hbm.at[pl.ds(r, RB), :], pbuf.at[b], sems_p[b])
        pltpu.async_copy(t_hbm.at[pl.ds(r, RB), :], tbuf.at[b], sems_t[b])

    def wait(b):
        pltpu.make_async_copy(p_hbm.at[pl.ds(row0, RB), :], pbuf.at[b],
                              sems_p[b]).wait()
        pltpu.make_async_copy(t_hbm.at[pl.ds(row0, RB), :], tbuf.at[b],
                              sems_t[b]).wait()

    def accum(p, t, s, c):
        m = t != 0.0
        d = jnp.where(m, p - t, 0.0)
        return s + d * d, c + jnp.where(m, 1.0, 0.0)

    def compute(b, accs):
        pb = pbuf.at[b]
        tb = tbuf.at[b]

        def row_body(r, accs):
            a = list(accs)
            for k in range(CHUNKS):
                off = k * L
                p = pb[r, pl.ds(off, L)]
                t = tb[r, pl.ds(off, L)]
                u = k % CU
                a[2 * u], a[2 * u + 1] = accum(p, t, a[2 * u], a[2 * u + 1])
            off = NCOLS - L
            p = pb[r, pl.ds(off, L)]
            t = jnp.where(edge_keep, tb[r, pl.ds(off, L)], 0.0)
            a[0], a[1] = accum(p, t, a[0], a[1])
            return tuple(a)

        return lax.fori_loop(0, RB, row_body, accs)

    zf = jnp.zeros((L,), jnp.float32)
    accs = (zf, zf) * CU

    start(0, 0)
    start(1, 1)

    def outer(g, accs):
        for b in range(2):
            wait(b)
            accs = compute(b, accs)
            start(2 * g + b + 2, b)
        return accs

    accs = lax.fori_loop(0, NBLK // 2 - 1, outer, accs)
    for b in range(2):
        wait(b)
        accs = compute(b, accs)

    s = accs[0]
    c = accs[1]
    for u in range(1, CU):
        s = s + accs[2 * u]
        c = c + accs[2 * u + 1]
    svec[...] = s
    cvec[...] = c
    pltpu.sync_copy(svec, sums_hbm.at[0, pl.ds(wid * L, L)])
    pltpu.sync_copy(cvec, cnts_hbm.at[0, pl.ds(wid * L, L)])


def _finish_body(sums_ref, cnts_ref, out_ref):
    loss = jnp.sum(sums_ref[...]) / jnp.sum(cnts_ref[...])
    out_ref[...] = loss.reshape(1, 1)


def kernel(predicted, target):
    sums, cnts = _sc_partials(predicted, target)
    loss = pl.pallas_call(
        _finish_body,
        out_shape=jax.ShapeDtypeStruct((1, 1), jnp.float32),
    )(sums, cnts)
    return loss[0, 0]


# speedup vs baseline: 2.5618x; 2.5618x over previous
import functools

import jax
import jax.numpy as jnp
from jax import lax
from jax.experimental import pallas as pl
from jax.experimental.pallas import tpu as pltpu
from jax.experimental.pallas import tpu_sc as plsc

NC, NS, L = 2, 16, 16
NW = NC * NS
NROWS, NCOLS = 16384, 1000
TC_ROWS = 12288
SC_ROWS = NROWS - TC_ROWS
ROWS_PER_W = SC_ROWS // NW
RB = 16
NBLK = ROWS_PER_W // RB
CHUNKS = NCOLS // L
CU = 4
TC_BLOCK = 1024

_mesh = plsc.VectorSubcoreMesh(core_axis_name="c", subcore_axis_name="s")


@functools.partial(
    pl.kernel,
    out_type=[
        jax.ShapeDtypeStruct((1, NW * L), jnp.float32),
        jax.ShapeDtypeStruct((1, NW * L), jnp.float32),
    ],
    mesh=_mesh,
    scratch_types=[
        pltpu.VMEM((2, RB, NCOLS), jnp.float32),
        pltpu.VMEM((2, RB, NCOLS), jnp.float32),
        pltpu.VMEM((L,), jnp.float32),
        pltpu.VMEM((L,), jnp.float32),
        pltpu.SemaphoreType.DMA,
        pltpu.SemaphoreType.DMA,
        pltpu.SemaphoreType.DMA,
        pltpu.SemaphoreType.DMA,
    ],
)
def _sc_partials(p_hbm, t_hbm, sums_hbm, cnts_hbm,
                 pbuf, tbuf, svec, cvec, sp0, sp1, st0, st1):
    wid = lax.axis_index("s") * NC + lax.axis_index("c")
    row0 = TC_ROWS + wid * ROWS_PER_W
    sems_p = (sp0, sp1)
    sems_t = (st0, st1)
    lane = lax.iota(jnp.int32, L)
    edge_keep = lane >= (L - (NCOLS - CHUNKS * L))

    def start(i, b):
        r = row0 + i * RB
        pltpu.async_copy(p_hbm.at[pl.ds(r, RB), :], pbuf.at[b], sems_p[b])
        pltpu.async_copy(t_hbm.at[pl.ds(r, RB), :], tbuf.at[b], sems_t[b])

    def wait(b):
        pltpu.make_async_copy(p_hbm.at[pl.ds(row0, RB), :], pbuf.at[b],
                              sems_p[b]).wait()
        pltpu.make_async_copy(t_hbm.at[pl.ds(row0, RB), :], tbuf.at[b],
                              sems_t[b]).wait()

    def accum(p, t, s, c):
        m = t != 0.0
        d = jnp.where(m, p - t, 0.0)
        return s + d * d, c + jnp.where(m, 1.0, 0.0)

    def compute(b, accs):
        pb = pbuf.at[b]
        tb = tbuf.at[b]

        def row_body(r, accs):
            def chunk_body(k, a):
                a = list(a)
                for u in range(CU):
                    off = (k * CU + u) * L
                    p = pb[r, pl.ds(off, L)]
                    t = tb[r, pl.ds(off, L)]
                    a[2 * u], a[2 * u + 1] = accum(p, t, a[2 * u],
                                                   a[2 * u + 1])
                return tuple(a)

            accs = lax.fori_loop(0, CHUNKS // CU, chunk_body, accs)
            off = NCOLS - L
            p = pb[r, pl.ds(off, L)]
            t = jnp.where(edge_keep, tb[r, pl.ds(off, L)], 0.0)
            s0, c0 = accum(p, t, accs[0], accs[1])
            return (s0, c0) + tuple(accs[2:])

        return lax.fori_loop(0, RB, row_body, accs)

    zf = jnp.zeros((L,), jnp.float32)
    accs = (zf, zf) * CU

    start(0, 0)
    start(1, 1)

    def outer(g, accs):
        for b in range(2):
            wait(b)
            accs = compute(b, accs)
            start(2 * g + b + 2, b)
        return accs

    accs = lax.fori_loop(0, NBLK // 2 - 1, outer, accs)
    for b in range(2):
        wait(b)
        accs = compute(b, accs)

    s = accs[0]
    c = accs[1]
    for u in range(1, CU):
        s = s + accs[2 * u]
        c = c + accs[2 * u + 1]
    svec[...] = s
    cvec[...] = c
    pltpu.sync_copy(svec, sums_hbm.at[0, pl.ds(wid * L, L)])
    pltpu.sync_copy(cvec, cnts_hbm.at[0, pl.ds(wid * L, L)])


def _tc_partials_body(p_ref, t_ref, s_ref, c_ref):
    i = pl.program_id(0)
    t = t_ref[...]
    m = t != 0.0
    d = jnp.where(m, p_ref[...] - t, 0.0)
    s = jnp.sum(d * d)
    c = jnp.sum(m.astype(jnp.float32))

    @pl.when(i == 0)
    def _():
        s_ref[0, 0] = 0.0
        c_ref[0, 0] = 0.0

    s_ref[0, 0] += s
    c_ref[0, 0] += c


def _tc_partials(predicted, target):
    grid = (TC_ROWS // TC_BLOCK,)
    return pl.pallas_call(
        _tc_partials_body,
        grid=grid,
        in_specs=[
            pl.BlockSpec((TC_BLOCK, NCOLS), lambda i: (i, 0)),
            pl.BlockSpec((TC_BLOCK, NCOLS), lambda i: (i, 0)),
        ],
        out_specs=[
            pl.BlockSpec(memory_space=pltpu.SMEM),
            pl.BlockSpec(memory_space=pltpu.SMEM),
        ],
        out_shape=[
            jax.ShapeDtypeStruct((1, 1), jnp.float32),
            jax.ShapeDtypeStruct((1, 1), jnp.float32),
        ],
    )(predicted, target)


def _finish_body(tcs_ref, tcc_ref, sums_ref, cnts_ref, out_ref):
    s = tcs_ref[0, 0] + jnp.sum(sums_ref[...])
    c = tcc_ref[0, 0] + jnp.sum(cnts_ref[...])
    out_ref[...] = (s / c).reshape(1, 1)


def kernel(predicted, target):
    sums, cnts = _sc_partials(predicted, target)
    tc_s, tc_c = _tc_partials(predicted, target)
    loss = pl.pallas_call(
        _finish_body,
        out_shape=jax.ShapeDtypeStruct((1, 1), jnp.float32),
    )(tc_s, tc_c, sums, cnts)
    return loss[0, 0]
